# baseline (speedup 1.0000x reference)
import functools

import jax
import jax.numpy as jnp
from jax import lax
from jax.experimental import pallas as pl
from jax.experimental.pallas import tpu as pltpu
from jax.experimental.pallas import tpu_sc as plsc

B = 1024
L = 200
D = 64
KW = 3
NLAYERS = 4
ROWS = B * L
HR = ROWS // 2

NC = 2
NS = 16
NW = NC * NS
CHH = 128
PER_W = HR // NW
CPW = PER_W // CHH


def _sc_gather_halves(table, idx_ab):
    mesh = plsc.VectorSubcoreMesh(core_axis_name="c", subcore_axis_name="s")

    @functools.partial(
        pl.kernel,
        out_type=jax.ShapeDtypeStruct((HR, 2 * D), jnp.float32),
        mesh=mesh,
        scratch_types=[
            pltpu.VMEM((CPW, CHH), jnp.int32),
            pltpu.VMEM((CPW, CHH), jnp.int32),
            pltpu.VMEM((CHH, D), jnp.float32),
            pltpu.VMEM((CHH, D), jnp.float32),
            pltpu.VMEM((CHH, D), jnp.float32),
            pltpu.VMEM((CHH, D), jnp.float32),
            pltpu.SemaphoreType.DMA,
            pltpu.SemaphoreType.DMA,
            pltpu.SemaphoreType.DMA,
            pltpu.SemaphoreType.DMA,
        ],
        compiler_params=pltpu.CompilerParams(use_tc_tiling_on_sc=False),
    )
    def sc_gather(tab_hbm, iab_hbm, out_hbm, iav, ibv, ba0, bb0, ba1, bb1,
                  gsem0, gsem1, psem0, psem1):
        wid = lax.axis_index("s") * NC + lax.axis_index("c")
        base_row = wid * PER_W
        pltpu.sync_copy(iab_hbm.at[0, wid], iav)
        pltpu.sync_copy(iab_hbm.at[1, wid], ibv)

        def gstart(i, ba, bb, gsem):
            pltpu.make_async_copy(tab_hbm.at[iav.at[i]], ba, gsem).start()
            pltpu.make_async_copy(tab_hbm.at[ibv.at[i]], bb, gsem).start()

        def gwait(ba, bb, gsem):
            pltpu.make_async_copy(tab_hbm.at[iav.at[0]], ba, gsem).wait()
            pltpu.make_async_copy(tab_hbm.at[ibv.at[0]], bb, gsem).wait()

        def pstart(i, ba, bb, psem):
            row0 = base_row + i * CHH
            pltpu.make_async_copy(
                ba, out_hbm.at[pl.ds(row0, CHH), pl.ds(0, D)], psem
            ).start()
            pltpu.make_async_copy(
                bb, out_hbm.at[pl.ds(row0, CHH), pl.ds(D, D)], psem
            ).start()

        def pwait(ba, bb, psem):
            pltpu.make_async_copy(
                ba, out_hbm.at[pl.ds(base_row, CHH), pl.ds(0, D)], psem
            ).wait()
            pltpu.make_async_copy(
                bb, out_hbm.at[pl.ds(base_row, CHH), pl.ds(D, D)], psem
            ).wait()

        gstart(0, ba0, bb0, gsem0)
        gstart(1, ba1, bb1, gsem1)

        def body(j, carry):
            i0 = 2 * j
            gwait(ba0, bb0, gsem0)
            pstart(i0, ba0, bb0, psem0)
            gwait(ba1, bb1, gsem1)
            pstart(i0 + 1, ba1, bb1, psem1)

            @pl.when(j + 1 < CPW // 2)
            def _():
                pwait(ba0, bb0, psem0)
                gstart(i0 + 2, ba0, bb0, gsem0)
                pwait(ba1, bb1, psem1)
                gstart(i0 + 3, ba1, bb1, gsem1)

            return carry

        lax.fori_loop(0, CPW // 2, body, 0)
        pwait(ba0, bb0, psem0)
        pwait(ba1, bb1, psem1)
        gstart(CPW - 1, ba0, bb0, gsem0)
        gwait(ba0, bb0, gsem0)
        pstart(CPW - 1, ba0, bb0, psem0)
        pwait(ba0, bb0, psem0)

    return sc_gather(table, idx_ab)


def _tap_weights(w):
    z = jnp.zeros((D, D), w.dtype)
    blocks = []
    for k in range(KW):
        top = jnp.concatenate([w[k], z], axis=1)
        bot = jnp.concatenate([z, w[k]], axis=1)
        blocks.append(jnp.concatenate([top, bot], axis=0))
    return jnp.concatenate(blocks, axis=0)


def _conv_body_half(w_ref, x_ref, o_ref, *, nb):
    m = nb * L // 2
    x = x_ref[...]
    l = lax.broadcasted_iota(jnp.int32, (m, 1), 0) % L
    not_first = l != 0
    not_last = l != (L - 1)
    zrow = jnp.zeros((1, 2 * D), jnp.bfloat16)
    zero = jnp.zeros((), jnp.bfloat16)
    zrowf = jnp.zeros((1, 2 * D), jnp.float32)
    zerof = jnp.zeros((), jnp.float32)
    for i in range(NLAYERS):
        xmf = jnp.where(not_first, jnp.concatenate([zrowf, x[: m - 1]], axis=0), zerof)
        xpf = jnp.where(not_last, jnp.concatenate([x[1:], zrowf], axis=0), zerof)
        xc = jnp.concatenate([xmf, x, xpf], axis=1).astype(jnp.bfloat16)
        y = lax.dot_general(
            xc, w_ref[i], (((1,), (0,)), ((), ())),
            preferred_element_type=jnp.float32,
        )
        x = jnp.maximum(y, 0.0)
    ha = x[:, :D].reshape(1, nb // 2, L, D)
    hb = x[:, D:].reshape(1, nb // 2, L, D)
    o_ref[...] = jnp.concatenate([ha, hb], axis=0)


def _conv_stack_half(x2, wt, nb=16, interpret=False):
    grid = (B // nb,)
    return pl.pallas_call(
        functools.partial(_conv_body_half, nb=nb),
        grid=grid,
        in_specs=[
            pl.BlockSpec((NLAYERS, 6 * D, 2 * D), lambda i: (0, 0, 0)),
            pl.BlockSpec((nb * L // 2, 2 * D), lambda i: (i, 0)),
        ],
        out_specs=pl.BlockSpec((2, nb // 2, L, D), lambda i: (0, i, 0, 0)),
        out_shape=jax.ShapeDtypeStruct((2, B // 2, L, D), jnp.float32),
        interpret=interpret,
    )(wt, x2)


def kernel(indices, table, w0, w1, w2, w3):
    idx_ab = indices.astype(jnp.int32).reshape(2, NW, CPW, CHH)
    x2 = _sc_gather_halves(table, idx_ab)
    wt = jnp.stack(
        [_tap_weights(w.reshape(KW, D, D)) for w in (w0, w1, w2, w3)]
    ).astype(jnp.bfloat16)
    out4 = _conv_stack_half(x2, wt)
    return out4.reshape(B, L, D)


# speedup vs baseline: 1.0291x; 1.0291x over previous
import functools

import jax
import jax.numpy as jnp
from jax import lax
from jax.experimental import pallas as pl
from jax.experimental.pallas import tpu as pltpu
from jax.experimental.pallas import tpu_sc as plsc

B = 1024
L = 200
D = 64
KW = 3
NLAYERS = 4
ROWS = B * L
HR = ROWS // 2

NC = 2
NS = 16
NW = NC * NS
CHH = 128
PER_W = HR // NW
CPW = PER_W // CHH


def _sc_gather_halves(table, idx_ab):
    mesh = plsc.VectorSubcoreMesh(core_axis_name="c", subcore_axis_name="s")

    @functools.partial(
        pl.kernel,
        out_type=jax.ShapeDtypeStruct((HR, 2 * D), jnp.float32),
        mesh=mesh,
        scratch_types=[
            pltpu.VMEM((CPW, CHH), jnp.int32),
            pltpu.VMEM((CPW, CHH), jnp.int32),
            pltpu.VMEM((CHH, D), jnp.float32),
            pltpu.VMEM((CHH, D), jnp.float32),
            pltpu.VMEM((CHH, D), jnp.float32),
            pltpu.VMEM((CHH, D), jnp.float32),
            pltpu.SemaphoreType.DMA,
            pltpu.SemaphoreType.DMA,
            pltpu.SemaphoreType.DMA,
            pltpu.SemaphoreType.DMA,
        ],
        compiler_params=pltpu.CompilerParams(use_tc_tiling_on_sc=False),
    )
    def sc_gather(tab_hbm, iab_hbm, out_hbm, iav, ibv, ba0, bb0, ba1, bb1,
                  gsem0, gsem1, psem0, psem1):
        wid = lax.axis_index("s") * NC + lax.axis_index("c")
        base_row = wid * PER_W
        pltpu.sync_copy(iab_hbm.at[0, wid], iav)
        pltpu.sync_copy(iab_hbm.at[1, wid], ibv)

        def gstart(i, ba, bb, gsem):
            pltpu.make_async_copy(tab_hbm.at[iav.at[i]], ba, gsem).start()
            pltpu.make_async_copy(tab_hbm.at[ibv.at[i]], bb, gsem).start()

        def gwait(ba, bb, gsem):
            pltpu.make_async_copy(tab_hbm.at[iav.at[0]], ba, gsem).wait()
            pltpu.make_async_copy(tab_hbm.at[ibv.at[0]], bb, gsem).wait()

        def pstart(i, ba, bb, psem):
            row0 = base_row + i * CHH
            pltpu.make_async_copy(
                ba, out_hbm.at[pl.ds(row0, CHH), pl.ds(0, D)], psem
            ).start()
            pltpu.make_async_copy(
                bb, out_hbm.at[pl.ds(row0, CHH), pl.ds(D, D)], psem
            ).start()

        def pwait(ba, bb, psem):
            pltpu.make_async_copy(
                ba, out_hbm.at[pl.ds(base_row, CHH), pl.ds(0, D)], psem
            ).wait()
            pltpu.make_async_copy(
                bb, out_hbm.at[pl.ds(base_row, CHH), pl.ds(D, D)], psem
            ).wait()

        gstart(0, ba0, bb0, gsem0)
        gstart(1, ba1, bb1, gsem1)

        def body(j, carry):
            i0 = 2 * j
            gwait(ba0, bb0, gsem0)
            pstart(i0, ba0, bb0, psem0)
            gwait(ba1, bb1, gsem1)
            pstart(i0 + 1, ba1, bb1, psem1)

            @pl.when(j + 1 < CPW // 2)
            def _():
                pwait(ba0, bb0, psem0)
                gstart(i0 + 2, ba0, bb0, gsem0)
                pwait(ba1, bb1, psem1)
                gstart(i0 + 3, ba1, bb1, gsem1)

            return carry

        lax.fori_loop(0, CPW // 2, body, 0)
        pwait(ba0, bb0, psem0)
        pwait(ba1, bb1, psem1)
        gstart(CPW - 1, ba0, bb0, gsem0)
        gwait(ba0, bb0, gsem0)
        pstart(CPW - 1, ba0, bb0, psem0)
        pwait(ba0, bb0, psem0)

    return sc_gather(table, idx_ab)


def _tap_weights(w):
    z = jnp.zeros((D, D), w.dtype)
    blocks = []
    for k in range(KW):
        top = jnp.concatenate([w[k], z], axis=1)
        bot = jnp.concatenate([z, w[k]], axis=1)
        blocks.append(jnp.concatenate([top, bot], axis=0))
    return jnp.concatenate(blocks, axis=0)


def _conv_body_half(w_ref, x_ref, o_ref, *, nb):
    m = nb * L // 2
    x = x_ref[...]
    l = lax.broadcasted_iota(jnp.int32, (m, 1), 0) % L
    not_first = l != 0
    not_last = l != (L - 1)
    zrow = jnp.zeros((1, 2 * D), jnp.bfloat16)
    zero = jnp.zeros((), jnp.bfloat16)
    zrowf = jnp.zeros((1, 2 * D), jnp.float32)
    zerof = jnp.zeros((), jnp.float32)
    for i in range(NLAYERS):
        xmf = jnp.where(not_first, jnp.concatenate([zrowf, x[: m - 1]], axis=0), zerof)
        xpf = jnp.where(not_last, jnp.concatenate([x[1:], zrowf], axis=0), zerof)
        xc = jnp.concatenate([xmf, x, xpf], axis=1).astype(jnp.bfloat16)
        y = lax.dot_general(
            xc, w_ref[i], (((1,), (0,)), ((), ())),
            preferred_element_type=jnp.float32,
        )
        x = jnp.maximum(y, 0.0)
    ha = x[:, :D].reshape(1, nb // 2, L, D)
    hb = x[:, D:].reshape(1, nb // 2, L, D)
    o_ref[...] = jnp.concatenate([ha, hb], axis=0)


def _conv_stack_half(x2, wt, nb=64, interpret=False):
    grid = (B // nb,)
    return pl.pallas_call(
        functools.partial(_conv_body_half, nb=nb),
        grid=grid,
        in_specs=[
            pl.BlockSpec((NLAYERS, 6 * D, 2 * D), lambda i: (0, 0, 0)),
            pl.BlockSpec((nb * L // 2, 2 * D), lambda i: (i, 0)),
        ],
        out_specs=pl.BlockSpec((2, nb // 2, L, D), lambda i: (0, i, 0, 0)),
        out_shape=jax.ShapeDtypeStruct((2, B // 2, L, D), jnp.float32),
        interpret=interpret,
    )(wt, x2)


def kernel(indices, table, w0, w1, w2, w3):
    idx_ab = indices.astype(jnp.int32).reshape(2, NW, CPW, CHH)
    x2 = _sc_gather_halves(table, idx_ab)
    wt = jnp.stack(
        [_tap_weights(w.reshape(KW, D, D)) for w in (w0, w1, w2, w3)]
    ).astype(jnp.bfloat16)
    out4 = _conv_stack_half(x2, wt)
    return out4.reshape(B, L, D)


# speedup vs baseline: 1.0352x; 1.0059x over previous
import functools

import jax
import jax.numpy as jnp
from jax import lax
from jax.experimental import pallas as pl
from jax.experimental.pallas import tpu as pltpu
from jax.experimental.pallas import tpu_sc as plsc

B = 1024
L = 200
D = 64
KW = 3
NLAYERS = 4
ROWS = B * L
HR = ROWS // 2

NC = 2
NS = 16
NW = NC * NS
CHH = 128
PER_W = HR // NW
CPW = PER_W // CHH


def _sc_gather_halves(table, idx_ab):
    mesh = plsc.VectorSubcoreMesh(core_axis_name="c", subcore_axis_name="s")

    @functools.partial(
        pl.kernel,
        out_type=jax.ShapeDtypeStruct((HR, 2 * D), jnp.float32),
        mesh=mesh,
        scratch_types=[
            pltpu.VMEM((CPW, CHH), jnp.int32),
            pltpu.VMEM((CPW, CHH), jnp.int32),
            pltpu.VMEM((CHH, D), jnp.float32),
            pltpu.VMEM((CHH, D), jnp.float32),
            pltpu.VMEM((CHH, D), jnp.float32),
            pltpu.VMEM((CHH, D), jnp.float32),
            pltpu.SemaphoreType.DMA,
            pltpu.SemaphoreType.DMA,
            pltpu.SemaphoreType.DMA,
            pltpu.SemaphoreType.DMA,
        ],
        compiler_params=pltpu.CompilerParams(use_tc_tiling_on_sc=False),
    )
    def sc_gather(tab_hbm, iab_hbm, out_hbm, iav, ibv, ba0, bb0, ba1, bb1,
                  gsem0, gsem1, psem0, psem1):
        wid = lax.axis_index("s") * NC + lax.axis_index("c")
        base_row = wid * PER_W
        pltpu.sync_copy(iab_hbm.at[0, wid], iav)
        pltpu.sync_copy(iab_hbm.at[1, wid], ibv)

        def gstart(i, ba, bb, gsem):
            pltpu.make_async_copy(tab_hbm.at[iav.at[i]], ba, gsem).start()
            pltpu.make_async_copy(tab_hbm.at[ibv.at[i]], bb, gsem).start()

        def gwait(ba, bb, gsem):
            pltpu.make_async_copy(tab_hbm.at[iav.at[0]], ba, gsem).wait()
            pltpu.make_async_copy(tab_hbm.at[ibv.at[0]], bb, gsem).wait()

        def pstart(i, ba, bb, psem):
            row0 = base_row + i * CHH
            pltpu.make_async_copy(
                ba, out_hbm.at[pl.ds(row0, CHH), pl.ds(0, D)], psem
            ).start()
            pltpu.make_async_copy(
                bb, out_hbm.at[pl.ds(row0, CHH), pl.ds(D, D)], psem
            ).start()

        def pwait(ba, bb, psem):
            pltpu.make_async_copy(
                ba, out_hbm.at[pl.ds(base_row, CHH), pl.ds(0, D)], psem
            ).wait()
            pltpu.make_async_copy(
                bb, out_hbm.at[pl.ds(base_row, CHH), pl.ds(D, D)], psem
            ).wait()

        gstart(0, ba0, bb0, gsem0)
        gstart(1, ba1, bb1, gsem1)

        def body(j, carry):
            i0 = 2 * j
            gwait(ba0, bb0, gsem0)
            pstart(i0, ba0, bb0, psem0)
            gwait(ba1, bb1, gsem1)
            pstart(i0 + 1, ba1, bb1, psem1)

            @pl.when(j + 1 < CPW // 2)
            def _():
                pwait(ba0, bb0, psem0)
                gstart(i0 + 2, ba0, bb0, gsem0)
                pwait(ba1, bb1, psem1)
                gstart(i0 + 3, ba1, bb1, gsem1)

            return carry

        lax.fori_loop(0, CPW // 2, body, 0)
        pwait(ba0, bb0, psem0)
        pwait(ba1, bb1, psem1)
        gstart(CPW - 1, ba0, bb0, gsem0)
        gwait(ba0, bb0, gsem0)
        pstart(CPW - 1, ba0, bb0, psem0)
        pwait(ba0, bb0, psem0)

    return sc_gather(table, idx_ab)


def _tap_weights(w):
    z = jnp.zeros((D, D), w.dtype)
    blocks = []
    for k in range(KW):
        top = jnp.concatenate([w[k], z], axis=1)
        bot = jnp.concatenate([z, w[k]], axis=1)
        blocks.append(jnp.concatenate([top, bot], axis=0))
    return jnp.concatenate(blocks, axis=0)


def _conv_body_half(w_ref, x_ref, o_ref, *, nb):
    m = nb * L // 2
    x = x_ref[...]
    l = lax.broadcasted_iota(jnp.int32, (m, 1), 0) % L
    not_first = l != 0
    not_last = l != (L - 1)
    zrow = jnp.zeros((1, 2 * D), jnp.bfloat16)
    zero = jnp.zeros((), jnp.bfloat16)
    zrowf = jnp.zeros((1, 2 * D), jnp.float32)
    zerof = jnp.zeros((), jnp.float32)
    for i in range(NLAYERS):
        xmf = jnp.where(not_first, jnp.concatenate([zrowf, x[: m - 1]], axis=0), zerof)
        xpf = jnp.where(not_last, jnp.concatenate([x[1:], zrowf], axis=0), zerof)
        xc = jnp.concatenate([xmf, x, xpf], axis=1).astype(jnp.bfloat16)
        y = lax.dot_general(
            xc, w_ref[i], (((1,), (0,)), ((), ())),
            preferred_element_type=jnp.float32,
        )
        x = jnp.maximum(y, 0.0)
    ha = x[:, :D].reshape(1, nb // 2, L, D)
    hb = x[:, D:].reshape(1, nb // 2, L, D)
    o_ref[...] = jnp.concatenate([ha, hb], axis=0)


def _conv_stack_half(x2, wt, nb=128, interpret=False):
    grid = (B // nb,)
    return pl.pallas_call(
        functools.partial(_conv_body_half, nb=nb),
        grid=grid,
        in_specs=[
            pl.BlockSpec((NLAYERS, 6 * D, 2 * D), lambda i: (0, 0, 0)),
            pl.BlockSpec((nb * L // 2, 2 * D), lambda i: (i, 0)),
        ],
        out_specs=pl.BlockSpec((2, nb // 2, L, D), lambda i: (0, i, 0, 0)),
        out_shape=jax.ShapeDtypeStruct((2, B // 2, L, D), jnp.float32),
        interpret=interpret,
    )(wt, x2)


def kernel(indices, table, w0, w1, w2, w3):
    idx_ab = indices.astype(jnp.int32).reshape(2, NW, CPW, CHH)
    x2 = _sc_gather_halves(table, idx_ab)
    wt = jnp.stack(
        [_tap_weights(w.reshape(KW, D, D)) for w in (w0, w1, w2, w3)]
    ).astype(jnp.bfloat16)
    out4 = _conv_stack_half(x2, wt)
    return out4.reshape(B, L, D)
